# baseline (speedup 1.0000x reference)
import functools

import jax
import jax.numpy as jnp
import numpy as np
from jax import lax
from jax.experimental import pallas as pl
from jax.experimental.pallas import tpu as pltpu
from jax.experimental.pallas import tpu_sc as plsc

N = 10000
E = 320000
D_IN = 128
H1 = 64
H2 = 32
LAT = 128
NCLS = 2
NG = 16

NCORE = 2
NSUB = 16
LANES = 16
NW = NCORE * NSUB
EW = E // NW
CHUNK = 128
EWP = 10240
NCH = EWP // CHUNK
PAD = 16
NP = N + PAD
RPT = N // NSUB
NBUF = 8


def _sc_mesh():
    return plsc.VectorSubcoreMesh(core_axis_name="c", subcore_axis_name="s")


_SC_PARAMS = dict(compiler_params=pltpu.CompilerParams(
    needs_layout_passes=False, use_tc_tiling_on_sc=False))


def _deg_body(dst_hbm, zeros_hbm, out_hbm, idx_v, deg_v):
    cid = lax.axis_index("c")
    sid = lax.axis_index("s")
    wid = sid * NCORE + cid
    pltpu.sync_copy(zeros_hbm, deg_v)
    pltpu.sync_copy(dst_hbm.at[wid], idx_v)
    ones = jnp.ones((LANES,), jnp.float32)

    def body(k, carry):
        for j in range(CHUNK // LANES):
            idx = idx_v[k, pl.ds(j * LANES, LANES)]
            plsc.addupdate_scatter(deg_v, [idx], ones)
        return carry

    lax.fori_loop(0, NCH, body, 0)
    pltpu.sync_copy(deg_v.at[pl.ds(0, N)], out_hbm.at[wid])


@functools.lru_cache(maxsize=None)
def _make_deg():
    return pl.kernel(
        _deg_body,
        out_type=jax.ShapeDtypeStruct((NW, N), jnp.float32),
        mesh=_sc_mesh(),
        scratch_types=[
            pltpu.VMEM((NCH, CHUNK), jnp.int32),
            pltpu.VMEM((NP,), jnp.float32),
        ],
        **_SC_PARAMS,
    )


def _prop_body(feat, table_hbm, src_hbm, dst_hbm, zeros_hbm, out_hbm,
               src_v, dst_v, b0, b1, b2, b3, b4, b5, b6, b7, acc,
               s0, s1, s2, s3, s4, s5, s6, s7):
    del feat
    bufs = (b0, b1, b2, b3, b4, b5, b6, b7)
    sems = (s0, s1, s2, s3, s4, s5, s6, s7)
    cid = lax.axis_index("c")
    sid = lax.axis_index("s")
    wid = sid * NCORE + cid
    pltpu.sync_copy(zeros_hbm.at[pl.ds(0, RPT)], acc.at[pl.ds(sid * RPT, RPT)])

    @pl.when(sid == NSUB - 1)
    def _():
        pltpu.sync_copy(zeros_hbm.at[pl.ds(0, PAD)], acc.at[pl.ds(N, PAD)])

    pltpu.sync_copy(src_hbm.at[wid], src_v)
    pltpu.sync_copy(dst_hbm.at[wid], dst_v)
    plsc.subcore_barrier()

    for p in range(NBUF - 1):
        pltpu.async_copy(table_hbm.at[src_v.at[p]], bufs[p], sems[p])

    def outer(g, carry):
        for b in range(NBUF):
            k = g * NBUF + b
            pltpu.make_async_copy(table_hbm.at[src_v.at[k]],
                                  bufs[b], sems[b]).wait()
            nk = k + NBUF - 1
            slot = (b + NBUF - 1) % NBUF

            @pl.when(nk < NCH)
            def _():
                pltpu.async_copy(table_hbm.at[src_v.at[nk]],
                                 bufs[slot], sems[slot])

            pltpu.sync_copy(bufs[b], acc.at[dst_v.at[k]], add=True)
        return carry

    lax.fori_loop(0, NCH // NBUF, outer, 0)
    plsc.subcore_barrier()
    pltpu.sync_copy(acc.at[pl.ds(sid * RPT, RPT)],
                    out_hbm.at[cid, pl.ds(sid * RPT, RPT)])


@functools.lru_cache(maxsize=None)
def _make_prop(feat):
    return pl.kernel(
        functools.partial(_prop_body, feat),
        out_type=jax.ShapeDtypeStruct((NCORE, N, feat), jnp.float32),
        mesh=_sc_mesh(),
        scratch_types=[
            pltpu.VMEM((NCH, CHUNK), jnp.int32),
            pltpu.VMEM((NCH, CHUNK), jnp.int32),
        ] + [pltpu.VMEM((CHUNK, feat), jnp.float32) for _ in range(NBUF)]
          + [pltpu.VMEM_SHARED((NP, feat), jnp.float32)]
          + [pltpu.SemaphoreType.DMA for _ in range(NBUF)],
        **_SC_PARAMS,
    )


def _tca1_body(x_ref, w1_ref, xw_ref):
    xw_ref[...] = jnp.dot(x_ref[...], w1_ref[...],
                          preferred_element_type=jnp.float32)


@functools.lru_cache(maxsize=None)
def _make_tca1():
    return pl.pallas_call(
        _tca1_body,
        out_shape=jax.ShapeDtypeStruct((N, H1), jnp.float32),
    )


def _tca2_body(degt_ref, xw_ref, hp_ref, dinv_ref):
    deg = jnp.sum(degt_ref[...], axis=1, keepdims=True) + 1.0
    dinv = lax.rsqrt(deg)
    hp_ref[pl.ds(0, N), :] = xw_ref[...] * dinv
    hp_ref[pl.ds(N, PAD), :] = jnp.zeros((PAD, H1), jnp.float32)
    dinv_ref[...] = dinv


@functools.lru_cache(maxsize=None)
def _make_tca2():
    return pl.pallas_call(
        _tca2_body,
        out_shape=(
            jax.ShapeDtypeStruct((NP, H1), jnp.float32),
            jax.ShapeDtypeStruct((N, 1), jnp.float32),
        ),
    )


def _bn_relu(y, g, b):
    m = jnp.mean(y, axis=0, keepdims=True)
    v = jnp.mean((y - m) ** 2, axis=0, keepdims=True)
    return jnp.maximum((y - m) * lax.rsqrt(v + 1e-5) * g + b, 0.0)


def _tcb_body(s1_ref, hp1_ref, dinv_ref, b1_ref, g1_ref, be1_ref, w2_ref,
              hp2_ref):
    dinv = dinv_ref[...]
    acc = s1_ref[0] + s1_ref[1] + hp1_ref[pl.ds(0, N), :]
    out1 = dinv * acc + b1_ref[...]
    h = _bn_relu(out1, g1_ref[...], be1_ref[...])
    xw2 = jnp.dot(h, w2_ref[...], preferred_element_type=jnp.float32)
    hp2_ref[pl.ds(0, N), :] = xw2 * dinv
    hp2_ref[pl.ds(N, PAD), :] = jnp.zeros((PAD, H2), jnp.float32)


@functools.lru_cache(maxsize=None)
def _make_tcb():
    return pl.pallas_call(
        _tcb_body,
        out_shape=jax.ShapeDtypeStruct((NP, H2), jnp.float32),
    )


def _tcc_body(s2_ref, hp2_ref, dinv_ref, b2_ref, g2_ref, be2_ref, batch_ref,
              eps_ref, wmu_ref, bmu_ref, wlv_ref, blv_ref, wd1_ref, bd1_ref,
              wd2_ref, bd2_ref, wd3_ref, bd3_ref, wc1_ref, bc1_ref, wc2_ref,
              bc2_ref, cls_ref, recon_ref, mu_ref, lv_ref, z_ref):
    dinv = dinv_ref[...]
    acc = s2_ref[0] + s2_ref[1] + hp2_ref[pl.ds(0, N), :]
    out2 = dinv * acc + b2_ref[...]
    h = _bn_relu(out2, g2_ref[...], be2_ref[...])

    gid = lax.broadcasted_iota(jnp.int32, (1, NG), 1)
    P = (batch_ref[...] == gid).astype(jnp.float32)
    dims = (((0,), (0,)), ((), ()))
    sums = lax.dot_general(P, h, dims, preferred_element_type=jnp.float32)
    cnt = lax.dot_general(P, jnp.ones((N, 1), jnp.float32), dims,
                          preferred_element_type=jnp.float32)
    hg = sums / jnp.maximum(cnt, 1.0)

    mu = jnp.dot(hg, wmu_ref[...], preferred_element_type=jnp.float32) + bmu_ref[...]
    lv = jnp.dot(hg, wlv_ref[...], preferred_element_type=jnp.float32) + blv_ref[...]
    std = jnp.exp(0.5 * lv)
    z = mu + eps_ref[...] * std
    d = jnp.maximum(jnp.dot(z, wd1_ref[...], preferred_element_type=jnp.float32) + bd1_ref[...], 0.0)
    d = jnp.maximum(jnp.dot(d, wd2_ref[...], preferred_element_type=jnp.float32) + bd2_ref[...], 0.0)
    recon = jnp.dot(d, wd3_ref[...], preferred_element_type=jnp.float32) + bd3_ref[...]
    c = jnp.maximum(jnp.dot(z, wc1_ref[...], preferred_element_type=jnp.float32) + bc1_ref[...], 0.0)
    cls = jnp.dot(c, wc2_ref[...], preferred_element_type=jnp.float32) + bc2_ref[...]

    cls_ref[...] = cls
    recon_ref[...] = recon
    mu_ref[...] = mu
    lv_ref[...] = lv
    z_ref[...] = z


@functools.lru_cache(maxsize=None)
def _make_tcc():
    return pl.pallas_call(
        _tcc_body,
        out_shape=(
            jax.ShapeDtypeStruct((NG, NCLS), jnp.float32),
            jax.ShapeDtypeStruct((NG, D_IN), jnp.float32),
            jax.ShapeDtypeStruct((NG, LAT), jnp.float32),
            jax.ShapeDtypeStruct((NG, LAT), jnp.float32),
            jax.ShapeDtypeStruct((NG, LAT), jnp.float32),
        ),
    )


def _pad_edges(row):
    pad = jnp.full((NW, EWP - EW), N, jnp.int32)
    return jnp.concatenate([row.reshape(NW, EW), pad], axis=1).reshape(
        NW, NCH, CHUNK)


def kernel(x, edge_index, batch, W1, b1, g1, be1, W2, b2, g2, be2, Wmu, bmu,
           Wlv, blv, Wd1, bd1, Wd2, bd2, Wd3, bd3, Wc1, bc1, Wc2, bc2):
    f32 = jnp.float32
    src = _pad_edges(edge_index[0])
    dst = _pad_edges(edge_index[1])

    deg_parts = _make_deg()(dst, jnp.zeros((NP,), f32))
    degt = deg_parts.T
    xw1 = _make_tca1()(x, W1)
    hp1, dinv = _make_tca2()(degt, xw1)

    s1 = _make_prop(H1)(hp1, src, dst, jnp.zeros((RPT, H1), f32))
    hp2 = _make_tcb()(s1, hp1, dinv, b1.reshape(1, H1),
                      g1.reshape(1, H1), be1.reshape(1, H1), W2)

    s2 = _make_prop(H2)(hp2, src, dst, jnp.zeros((RPT, H2), f32))

    cls, recon, mu, lv, z = _make_tcc()(
        s2, hp2, dinv, b2.reshape(1, H2), g2.reshape(1, H2),
        be2.reshape(1, H2), batch.reshape(N, 1),
        jax.random.normal(jax.random.key(42), (NG, LAT), f32),
        Wmu, bmu.reshape(1, LAT), Wlv, blv.reshape(1, LAT),
        Wd1, bd1.reshape(1, H2), Wd2, bd2.reshape(1, H2),
        Wd3, bd3.reshape(1, D_IN), Wc1, bc1.reshape(1, LAT // 2),
        Wc2, bc2.reshape(1, NCLS))
    return (cls, recon, mu, lv, z)


# speedup vs baseline: 1.7860x; 1.7860x over previous
import functools

import jax
import jax.numpy as jnp
import numpy as np
from jax import lax
from jax.experimental import pallas as pl
from jax.experimental.pallas import tpu as pltpu
from jax.experimental.pallas import tpu_sc as plsc

N = 10000
E = 320000
D_IN = 128
H1 = 64
H2 = 32
LAT = 128
NCLS = 2
NG = 16

NCORE = 2
NSUB = 16
LANES = 16
NW = NCORE * NSUB
EW = E // NW
CHUNK = 128
EWP = 10240
NCH = EWP // CHUNK
PAD = 16
NP = N + PAD
RPT = N // NSUB
NBUF = 8


def _sc_mesh():
    return plsc.VectorSubcoreMesh(core_axis_name="c", subcore_axis_name="s")


_SC_PARAMS = dict(compiler_params=pltpu.CompilerParams(
    needs_layout_passes=False, use_tc_tiling_on_sc=False))


def _deg_body(dst_hbm, zeros_hbm, out_hbm, idx_v, deg_v):
    cid = lax.axis_index("c")
    sid = lax.axis_index("s")
    wid = sid * NCORE + cid
    pltpu.sync_copy(zeros_hbm, deg_v)
    pltpu.sync_copy(dst_hbm.at[wid], idx_v)
    ones = jnp.ones((LANES,), jnp.float32)

    def body(k, carry):
        for j in range(CHUNK // LANES):
            idx = idx_v[k, pl.ds(j * LANES, LANES)]
            plsc.addupdate_scatter(deg_v, [idx], ones)
        return carry

    lax.fori_loop(0, NCH, body, 0)
    pltpu.sync_copy(deg_v.at[pl.ds(0, N)], out_hbm.at[wid])


@functools.lru_cache(maxsize=None)
def _make_deg():
    return pl.kernel(
        _deg_body,
        out_type=jax.ShapeDtypeStruct((NW, N), jnp.float32),
        mesh=_sc_mesh(),
        scratch_types=[
            pltpu.VMEM((NCH, CHUNK), jnp.int32),
            pltpu.VMEM((NP,), jnp.float32),
        ],
        **_SC_PARAMS,
    )


def _prop_body(feat, table_hbm, src_hbm, dst_hbm, zeros_hbm, out_hbm,
               src_v, dst_v, b0, b1, b2, b3, b4, b5, b6, b7, acc,
               s0, s1, s2, s3, s4, s5, s6, s7):
    del feat
    bufs = (b0, b1, b2, b3, b4, b5, b6, b7)
    sems = (s0, s1, s2, s3, s4, s5, s6, s7)
    cid = lax.axis_index("c")
    sid = lax.axis_index("s")
    wid = sid * NCORE + cid
    pltpu.sync_copy(zeros_hbm.at[pl.ds(0, RPT)], acc.at[pl.ds(sid * RPT, RPT)])

    @pl.when(sid == NSUB - 1)
    def _():
        pltpu.sync_copy(zeros_hbm.at[pl.ds(0, PAD)], acc.at[pl.ds(N, PAD)])

    pltpu.sync_copy(src_hbm.at[wid], src_v)
    pltpu.sync_copy(dst_hbm.at[wid], dst_v)
    plsc.subcore_barrier()

    for p in range(NBUF - 1):
        pltpu.async_copy(table_hbm.at[src_v.at[p]], bufs[p], sems[p])

    def outer(g, carry):
        for b in range(NBUF):
            k = g * NBUF + b
            pltpu.make_async_copy(table_hbm.at[src_v.at[k]],
                                  bufs[b], sems[b]).wait()
            nk = k + NBUF - 1
            slot = (b + NBUF - 1) % NBUF

            @pl.when(nk < NCH)
            def _():
                pltpu.async_copy(table_hbm.at[src_v.at[nk]],
                                 bufs[slot], sems[slot])

            pltpu.sync_copy(bufs[b], acc.at[dst_v.at[k]], add=True)
        return carry

    lax.fori_loop(0, NCH // NBUF, outer, 0)
    plsc.subcore_barrier()
    pltpu.sync_copy(acc.at[pl.ds(sid * RPT, RPT)],
                    out_hbm.at[cid, pl.ds(sid * RPT, RPT)])


@functools.lru_cache(maxsize=None)
def _make_prop(feat):
    return pl.kernel(
        functools.partial(_prop_body, feat),
        out_type=jax.ShapeDtypeStruct((NCORE, N, feat), jnp.float32),
        mesh=_sc_mesh(),
        scratch_types=[
            pltpu.VMEM((NCH, CHUNK), jnp.int32),
            pltpu.VMEM((NCH, CHUNK), jnp.int32),
        ] + [pltpu.VMEM((CHUNK, feat), jnp.float32) for _ in range(NBUF)]
          + [pltpu.VMEM_SHARED((NP, feat), jnp.float32)]
          + [pltpu.SemaphoreType.DMA for _ in range(NBUF)],
        **_SC_PARAMS,
    )


def _tca1_body(x_ref, w1_ref, xw_ref):
    xw_ref[...] = jnp.dot(x_ref[...], w1_ref[...],
                          preferred_element_type=jnp.float32)


@functools.lru_cache(maxsize=None)
def _make_tca1():
    return pl.pallas_call(
        _tca1_body,
        out_shape=jax.ShapeDtypeStruct((N, H1), jnp.float32),
    )


def _tca2_body(degt_ref, xw_ref, hp_ref, dinv_ref):
    deg = jnp.sum(degt_ref[...], axis=1, keepdims=True) + 1.0
    dinv = lax.rsqrt(deg)
    hp_ref[pl.ds(0, N), :] = xw_ref[...] * dinv
    hp_ref[pl.ds(N, PAD), :] = jnp.zeros((PAD, H1), jnp.float32)
    dinv_ref[...] = dinv


@functools.lru_cache(maxsize=None)
def _make_tca2():
    return pl.pallas_call(
        _tca2_body,
        out_shape=(
            jax.ShapeDtypeStruct((NP, H1), jnp.float32),
            jax.ShapeDtypeStruct((N, 1), jnp.float32),
        ),
    )


def _bn_relu(y, g, b):
    m = jnp.mean(y, axis=0, keepdims=True)
    v = jnp.mean((y - m) ** 2, axis=0, keepdims=True)
    return jnp.maximum((y - m) * lax.rsqrt(v + 1e-5) * g + b, 0.0)


def _tcb_body(s1_ref, hp1_ref, dinv_ref, b1_ref, g1_ref, be1_ref, w2_ref,
              hp2_ref):
    dinv = dinv_ref[...]
    acc = s1_ref[0] + s1_ref[1] + hp1_ref[pl.ds(0, N), :]
    out1 = dinv * acc + b1_ref[...]
    h = _bn_relu(out1, g1_ref[...], be1_ref[...])
    xw2 = jnp.dot(h, w2_ref[...], preferred_element_type=jnp.float32)
    hp2_ref[pl.ds(0, N), :] = xw2 * dinv
    hp2_ref[pl.ds(N, PAD), :] = jnp.zeros((PAD, H2), jnp.float32)


@functools.lru_cache(maxsize=None)
def _make_tcb():
    return pl.pallas_call(
        _tcb_body,
        out_shape=jax.ShapeDtypeStruct((NP, H2), jnp.float32),
    )


def _tcc_body(s2_ref, hp2_ref, dinv_ref, b2_ref, g2_ref, be2_ref, batch_ref,
              eps_ref, wmu_ref, bmu_ref, wlv_ref, blv_ref, wd1_ref, bd1_ref,
              wd2_ref, bd2_ref, wd3_ref, bd3_ref, wc1_ref, bc1_ref, wc2_ref,
              bc2_ref, cls_ref, recon_ref, mu_ref, lv_ref, z_ref):
    dinv = dinv_ref[...]
    acc = s2_ref[0] + s2_ref[1] + hp2_ref[pl.ds(0, N), :]
    out2 = dinv * acc + b2_ref[...]
    h = _bn_relu(out2, g2_ref[...], be2_ref[...])

    gid = lax.broadcasted_iota(jnp.int32, (1, NG), 1)
    P = (batch_ref[...] == gid).astype(jnp.float32)
    dims = (((0,), (0,)), ((), ()))
    sums = lax.dot_general(P, h, dims, preferred_element_type=jnp.float32)
    cnt = lax.dot_general(P, jnp.ones((N, 1), jnp.float32), dims,
                          preferred_element_type=jnp.float32)
    hg = sums / jnp.maximum(cnt, 1.0)

    mu = jnp.dot(hg, wmu_ref[...], preferred_element_type=jnp.float32) + bmu_ref[...]
    lv = jnp.dot(hg, wlv_ref[...], preferred_element_type=jnp.float32) + blv_ref[...]
    std = jnp.exp(0.5 * lv)
    z = mu + eps_ref[...] * std
    d = jnp.maximum(jnp.dot(z, wd1_ref[...], preferred_element_type=jnp.float32) + bd1_ref[...], 0.0)
    d = jnp.maximum(jnp.dot(d, wd2_ref[...], preferred_element_type=jnp.float32) + bd2_ref[...], 0.0)
    recon = jnp.dot(d, wd3_ref[...], preferred_element_type=jnp.float32) + bd3_ref[...]
    c = jnp.maximum(jnp.dot(z, wc1_ref[...], preferred_element_type=jnp.float32) + bc1_ref[...], 0.0)
    cls = jnp.dot(c, wc2_ref[...], preferred_element_type=jnp.float32) + bc2_ref[...]

    cls_ref[...] = cls
    recon_ref[...] = recon
    mu_ref[...] = mu
    lv_ref[...] = lv
    z_ref[...] = z


@functools.lru_cache(maxsize=None)
def _make_tcc():
    return pl.pallas_call(
        _tcc_body,
        out_shape=(
            jax.ShapeDtypeStruct((NG, NCLS), jnp.float32),
            jax.ShapeDtypeStruct((NG, D_IN), jnp.float32),
            jax.ShapeDtypeStruct((NG, LAT), jnp.float32),
            jax.ShapeDtypeStruct((NG, LAT), jnp.float32),
            jax.ShapeDtypeStruct((NG, LAT), jnp.float32),
        ),
    )


def _pad_edges(row):
    pad = N + jnp.tile(jnp.arange(PAD, dtype=jnp.int32),
                       (NW, (EWP - EW) // PAD))
    return jnp.concatenate([row.reshape(NW, EW), pad], axis=1).reshape(
        NW, NCH, CHUNK)


def kernel(x, edge_index, batch, W1, b1, g1, be1, W2, b2, g2, be2, Wmu, bmu,
           Wlv, blv, Wd1, bd1, Wd2, bd2, Wd3, bd3, Wc1, bc1, Wc2, bc2):
    f32 = jnp.float32
    src = _pad_edges(edge_index[0])
    dst = _pad_edges(edge_index[1])

    deg_parts = _make_deg()(dst, jnp.zeros((NP,), f32))
    degt = deg_parts.T
    xw1 = _make_tca1()(x, W1)
    hp1, dinv = _make_tca2()(degt, xw1)

    s1 = _make_prop(H1)(hp1, src, dst, jnp.zeros((RPT, H1), f32))
    hp2 = _make_tcb()(s1, hp1, dinv, b1.reshape(1, H1),
                      g1.reshape(1, H1), be1.reshape(1, H1), W2)

    s2 = _make_prop(H2)(hp2, src, dst, jnp.zeros((RPT, H2), f32))

    cls, recon, mu, lv, z = _make_tcc()(
        s2, hp2, dinv, b2.reshape(1, H2), g2.reshape(1, H2),
        be2.reshape(1, H2), batch.reshape(N, 1),
        jax.random.normal(jax.random.key(42), (NG, LAT), f32),
        Wmu, bmu.reshape(1, LAT), Wlv, blv.reshape(1, LAT),
        Wd1, bd1.reshape(1, H2), Wd2, bd2.reshape(1, H2),
        Wd3, bd3.reshape(1, D_IN), Wc1, bc1.reshape(1, LAT // 2),
        Wc2, bc2.reshape(1, NCLS))
    return (cls, recon, mu, lv, z)


# speedup vs baseline: 2.0608x; 1.1539x over previous
import functools

import jax
import jax.numpy as jnp
import numpy as np
from jax import lax
from jax.experimental import pallas as pl
from jax.experimental.pallas import tpu as pltpu
from jax.experimental.pallas import tpu_sc as plsc

N = 10000
E = 320000
D_IN = 128
H1 = 64
H2 = 32
LAT = 128
NCLS = 2
NG = 16

NCORE = 2
NSUB = 16
LANES = 16
NW = NCORE * NSUB
EW = E // NW
CHUNK = 80
NCH = EW // CHUNK
RPT = N // NSUB
NBUF = 5


def _sc_mesh():
    return plsc.VectorSubcoreMesh(core_axis_name="c", subcore_axis_name="s")


_SC_PARAMS = dict(compiler_params=pltpu.CompilerParams(
    needs_layout_passes=False, use_tc_tiling_on_sc=False))


def _deg_body(dst_hbm, zeros_hbm, out_hbm, idx_v, deg_v):
    cid = lax.axis_index("c")
    sid = lax.axis_index("s")
    wid = sid * NCORE + cid
    pltpu.sync_copy(zeros_hbm, deg_v)
    pltpu.sync_copy(dst_hbm.at[wid], idx_v)
    ones = jnp.ones((LANES,), jnp.float32)

    def body(k, carry):
        for j in range(CHUNK // LANES):
            idx = idx_v[k, pl.ds(j * LANES, LANES)]
            plsc.addupdate_scatter(deg_v, [idx], ones)
        return carry

    lax.fori_loop(0, NCH, body, 0)
    pltpu.sync_copy(deg_v, out_hbm.at[wid])


@functools.lru_cache(maxsize=None)
def _make_deg():
    return pl.kernel(
        _deg_body,
        out_type=jax.ShapeDtypeStruct((NW, N), jnp.float32),
        mesh=_sc_mesh(),
        scratch_types=[
            pltpu.VMEM((NCH, CHUNK), jnp.int32),
            pltpu.VMEM((N,), jnp.float32),
        ],
        **_SC_PARAMS,
    )


def _prop_body(feat, table_hbm, src_hbm, dst_hbm, zeros_hbm, out_hbm,
               src_v, dst_v, b0, b1, b2, b3, b4, acc,
               s0, s1, s2, s3, s4):
    del feat
    bufs = (b0, b1, b2, b3, b4)
    sems = (s0, s1, s2, s3, s4)
    cid = lax.axis_index("c")
    sid = lax.axis_index("s")
    wid = sid * NCORE + cid
    pltpu.sync_copy(zeros_hbm, acc.at[pl.ds(sid * RPT, RPT)])
    pltpu.sync_copy(src_hbm.at[wid], src_v)
    pltpu.sync_copy(dst_hbm.at[wid], dst_v)
    plsc.subcore_barrier()

    for p in range(NBUF - 1):
        pltpu.async_copy(table_hbm.at[src_v.at[p]], bufs[p], sems[p])

    def outer(g, carry):
        for b in range(NBUF):
            k = g * NBUF + b
            pltpu.make_async_copy(table_hbm.at[src_v.at[k]],
                                  bufs[b], sems[b]).wait()
            nk = k + NBUF - 1
            slot = (b + NBUF - 1) % NBUF

            @pl.when(nk < NCH)
            def _():
                pltpu.async_copy(table_hbm.at[src_v.at[nk]],
                                 bufs[slot], sems[slot])

            pltpu.sync_copy(bufs[b], acc.at[dst_v.at[k]], add=True)
        return carry

    lax.fori_loop(0, NCH // NBUF, outer, 0)
    plsc.subcore_barrier()
    pltpu.sync_copy(acc.at[pl.ds(sid * RPT, RPT)],
                    out_hbm.at[cid, pl.ds(sid * RPT, RPT)])


@functools.lru_cache(maxsize=None)
def _make_prop(feat):
    return pl.kernel(
        functools.partial(_prop_body, feat),
        out_type=jax.ShapeDtypeStruct((NCORE, N, feat), jnp.float32),
        mesh=_sc_mesh(),
        scratch_types=[
            pltpu.VMEM((NCH, CHUNK), jnp.int32),
            pltpu.VMEM((NCH, CHUNK), jnp.int32),
        ] + [pltpu.VMEM((CHUNK, feat), jnp.float32) for _ in range(NBUF)]
          + [pltpu.VMEM_SHARED((N, feat), jnp.float32)]
          + [pltpu.SemaphoreType.DMA for _ in range(NBUF)],
        **_SC_PARAMS,
    )


def _tca1_body(x_ref, w1_ref, xw_ref):
    xw_ref[...] = jnp.dot(x_ref[...], w1_ref[...],
                          preferred_element_type=jnp.float32)


@functools.lru_cache(maxsize=None)
def _make_tca1():
    return pl.pallas_call(
        _tca1_body,
        out_shape=jax.ShapeDtypeStruct((N, H1), jnp.float32),
    )


def _tca2_body(degp_ref, xw_ref, hp_ref, dinv_ref):
    deg = lax.dot_general(degp_ref[...], jnp.ones((NW, 1), jnp.float32),
                          (((0,), (0,)), ((), ())),
                          preferred_element_type=jnp.float32) + 1.0
    dinv = lax.rsqrt(deg)
    hp_ref[...] = xw_ref[...] * dinv
    dinv_ref[...] = dinv


@functools.lru_cache(maxsize=None)
def _make_tca2():
    return pl.pallas_call(
        _tca2_body,
        out_shape=(
            jax.ShapeDtypeStruct((N, H1), jnp.float32),
            jax.ShapeDtypeStruct((N, 1), jnp.float32),
        ),
    )


def _bn_relu(y, g, b):
    m = jnp.mean(y, axis=0, keepdims=True)
    v = jnp.mean((y - m) ** 2, axis=0, keepdims=True)
    return jnp.maximum((y - m) * lax.rsqrt(v + 1e-5) * g + b, 0.0)


def _tcb_body(s1_ref, hp1_ref, dinv_ref, b1_ref, g1_ref, be1_ref, w2_ref,
              hp2_ref):
    dinv = dinv_ref[...]
    acc = s1_ref[0] + s1_ref[1] + hp1_ref[...]
    out1 = dinv * acc + b1_ref[...]
    h = _bn_relu(out1, g1_ref[...], be1_ref[...])
    xw2 = jnp.dot(h, w2_ref[...], preferred_element_type=jnp.float32)
    hp2_ref[...] = xw2 * dinv


@functools.lru_cache(maxsize=None)
def _make_tcb():
    return pl.pallas_call(
        _tcb_body,
        out_shape=jax.ShapeDtypeStruct((N, H2), jnp.float32),
    )


def _tcc_body(s2_ref, hp2_ref, dinv_ref, b2_ref, g2_ref, be2_ref, batch_ref,
              eps_ref, wmu_ref, bmu_ref, wlv_ref, blv_ref, wd1_ref, bd1_ref,
              wd2_ref, bd2_ref, wd3_ref, bd3_ref, wc1_ref, bc1_ref, wc2_ref,
              bc2_ref, cls_ref, recon_ref, mu_ref, lv_ref, z_ref):
    dinv = dinv_ref[...]
    acc = s2_ref[0] + s2_ref[1] + hp2_ref[...]
    out2 = dinv * acc + b2_ref[...]
    h = _bn_relu(out2, g2_ref[...], be2_ref[...])

    gid = lax.broadcasted_iota(jnp.int32, (1, NG), 1)
    P = (batch_ref[...] == gid).astype(jnp.float32)
    dims = (((0,), (0,)), ((), ()))
    sums = lax.dot_general(P, h, dims, preferred_element_type=jnp.float32)
    cnt = lax.dot_general(P, jnp.ones((N, 1), jnp.float32), dims,
                          preferred_element_type=jnp.float32)
    hg = sums / jnp.maximum(cnt, 1.0)

    mu = jnp.dot(hg, wmu_ref[...], preferred_element_type=jnp.float32) + bmu_ref[...]
    lv = jnp.dot(hg, wlv_ref[...], preferred_element_type=jnp.float32) + blv_ref[...]
    std = jnp.exp(0.5 * lv)
    z = mu + eps_ref[...] * std
    d = jnp.maximum(jnp.dot(z, wd1_ref[...], preferred_element_type=jnp.float32) + bd1_ref[...], 0.0)
    d = jnp.maximum(jnp.dot(d, wd2_ref[...], preferred_element_type=jnp.float32) + bd2_ref[...], 0.0)
    recon = jnp.dot(d, wd3_ref[...], preferred_element_type=jnp.float32) + bd3_ref[...]
    c = jnp.maximum(jnp.dot(z, wc1_ref[...], preferred_element_type=jnp.float32) + bc1_ref[...], 0.0)
    cls = jnp.dot(c, wc2_ref[...], preferred_element_type=jnp.float32) + bc2_ref[...]

    cls_ref[...] = cls
    recon_ref[...] = recon
    mu_ref[...] = mu
    lv_ref[...] = lv
    z_ref[...] = z


@functools.lru_cache(maxsize=None)
def _make_tcc():
    return pl.pallas_call(
        _tcc_body,
        out_shape=(
            jax.ShapeDtypeStruct((NG, NCLS), jnp.float32),
            jax.ShapeDtypeStruct((NG, D_IN), jnp.float32),
            jax.ShapeDtypeStruct((NG, LAT), jnp.float32),
            jax.ShapeDtypeStruct((NG, LAT), jnp.float32),
            jax.ShapeDtypeStruct((NG, LAT), jnp.float32),
        ),
    )


def _shape_edges(row):
    return row.reshape(NW, NCH, CHUNK)


def kernel(x, edge_index, batch, W1, b1, g1, be1, W2, b2, g2, be2, Wmu, bmu,
           Wlv, blv, Wd1, bd1, Wd2, bd2, Wd3, bd3, Wc1, bc1, Wc2, bc2):
    f32 = jnp.float32
    src = _shape_edges(edge_index[0])
    dst = _shape_edges(edge_index[1])

    deg_parts = _make_deg()(dst, jnp.zeros((N,), f32))
    xw1 = _make_tca1()(x, W1)
    hp1, dinv = _make_tca2()(deg_parts, xw1)

    s1 = _make_prop(H1)(hp1, src, dst, jnp.zeros((RPT, H1), f32))
    hp2 = _make_tcb()(s1, hp1, dinv, b1.reshape(1, H1),
                      g1.reshape(1, H1), be1.reshape(1, H1), W2)

    s2 = _make_prop(H2)(hp2, src, dst, jnp.zeros((RPT, H2), f32))

    cls, recon, mu, lv, z = _make_tcc()(
        s2, hp2, dinv, b2.reshape(1, H2), g2.reshape(1, H2),
        be2.reshape(1, H2), batch.reshape(N, 1),
        jax.random.normal(jax.random.key(42), (NG, LAT), f32),
        Wmu, bmu.reshape(1, LAT), Wlv, blv.reshape(1, LAT),
        Wd1, bd1.reshape(1, H2), Wd2, bd2.reshape(1, H2),
        Wd3, bd3.reshape(1, D_IN), Wc1, bc1.reshape(1, LAT // 2),
        Wc2, bc2.reshape(1, NCLS))
    return (cls, recon, mu, lv, z)


# speedup vs baseline: 2.1169x; 1.0272x over previous
import functools

import jax
import jax.numpy as jnp
import numpy as np
from jax import lax
from jax.experimental import pallas as pl
from jax.experimental.pallas import tpu as pltpu
from jax.experimental.pallas import tpu_sc as plsc

N = 10000
E = 320000
D_IN = 128
H1 = 64
H2 = 32
LAT = 128
NCLS = 2
NG = 16

NCORE = 2
NSUB = 16
LANES = 16
NW = NCORE * NSUB
EW = E // NW
CHUNK = 80
NCH = EW // CHUNK
RPT = N // NSUB
NBUF = 5


def _sc_mesh():
    return plsc.VectorSubcoreMesh(core_axis_name="c", subcore_axis_name="s")


_SC_PARAMS = dict(compiler_params=pltpu.CompilerParams(
    needs_layout_passes=False, use_tc_tiling_on_sc=False))


def _deg_body(dst_hbm, zeros_hbm, out_hbm, idx_v, deg_v):
    cid = lax.axis_index("c")
    sid = lax.axis_index("s")
    wid = sid * NCORE + cid
    pltpu.sync_copy(zeros_hbm, deg_v)
    pltpu.sync_copy(dst_hbm.at[wid], idx_v)
    ones = jnp.ones((LANES,), jnp.float32)

    def body(k, carry):
        for j in range(CHUNK // LANES):
            idx = idx_v[k, pl.ds(j * LANES, LANES)]
            plsc.addupdate_scatter(deg_v, [idx], ones)
        return carry

    lax.fori_loop(0, NCH, body, 0)
    pltpu.sync_copy(deg_v, out_hbm.at[wid])


@functools.lru_cache(maxsize=None)
def _make_deg():
    return pl.kernel(
        _deg_body,
        out_type=jax.ShapeDtypeStruct((NW, N), jnp.float32),
        mesh=_sc_mesh(),
        scratch_types=[
            pltpu.VMEM((NCH, CHUNK), jnp.int32),
            pltpu.VMEM((N,), jnp.float32),
        ],
        **_SC_PARAMS,
    )


def _prop_body(feat, table_hbm, src_hbm, dst_hbm, zeros_hbm, out_hbm,
               src_v, dst_v, b0, b1, b2, b3, b4, acc,
               s0, s1, s2, s3, s4):
    del feat
    bufs = (b0, b1, b2, b3, b4)
    sems = (s0, s1, s2, s3, s4)
    cid = lax.axis_index("c")
    sid = lax.axis_index("s")
    wid = sid * NCORE + cid
    @pl.when(cid == 0)
    def _():
        pltpu.sync_copy(table_hbm.at[pl.ds(sid * RPT, RPT)],
                        acc.at[pl.ds(sid * RPT, RPT)])

    @pl.when(cid == 1)
    def _():
        pltpu.sync_copy(zeros_hbm, acc.at[pl.ds(sid * RPT, RPT)])

    pltpu.sync_copy(src_hbm.at[wid], src_v)
    pltpu.sync_copy(dst_hbm.at[wid], dst_v)
    plsc.subcore_barrier()

    for p in range(NBUF - 1):
        pltpu.async_copy(table_hbm.at[src_v.at[p]], bufs[p], sems[p])

    def outer(g, carry):
        for b in range(NBUF):
            k = g * NBUF + b
            pltpu.make_async_copy(table_hbm.at[src_v.at[k]],
                                  bufs[b], sems[b]).wait()
            nk = k + NBUF - 1
            slot = (b + NBUF - 1) % NBUF

            @pl.when(nk < NCH)
            def _():
                pltpu.async_copy(table_hbm.at[src_v.at[nk]],
                                 bufs[slot], sems[slot])

            pltpu.sync_copy(bufs[b], acc.at[dst_v.at[k]], add=True)
        return carry

    lax.fori_loop(0, NCH // NBUF, outer, 0)
    plsc.subcore_barrier()
    pltpu.sync_copy(acc.at[pl.ds(sid * RPT, RPT)],
                    out_hbm.at[cid, pl.ds(sid * RPT, RPT)])


@functools.lru_cache(maxsize=None)
def _make_prop(feat):
    return pl.kernel(
        functools.partial(_prop_body, feat),
        out_type=jax.ShapeDtypeStruct((NCORE, N, feat), jnp.float32),
        mesh=_sc_mesh(),
        scratch_types=[
            pltpu.VMEM((NCH, CHUNK), jnp.int32),
            pltpu.VMEM((NCH, CHUNK), jnp.int32),
        ] + [pltpu.VMEM((CHUNK, feat), jnp.float32) for _ in range(NBUF)]
          + [pltpu.VMEM_SHARED((N, feat), jnp.float32)]
          + [pltpu.SemaphoreType.DMA for _ in range(NBUF)],
        **_SC_PARAMS,
    )


def _tca1_body(x_ref, w1_ref, xw_ref):
    xw_ref[...] = jnp.dot(x_ref[...], w1_ref[...],
                          preferred_element_type=jnp.float32)


@functools.lru_cache(maxsize=None)
def _make_tca1():
    return pl.pallas_call(
        _tca1_body,
        out_shape=jax.ShapeDtypeStruct((N, H1), jnp.float32),
    )


def _tca2_body(degp_ref, xw_ref, hp_ref, dinv_ref):
    deg = lax.dot_general(degp_ref[...], jnp.ones((NW, 1), jnp.float32),
                          (((0,), (0,)), ((), ())),
                          preferred_element_type=jnp.float32) + 1.0
    dinv = lax.rsqrt(deg)
    hp_ref[...] = xw_ref[...] * dinv
    dinv_ref[...] = dinv


@functools.lru_cache(maxsize=None)
def _make_tca2():
    return pl.pallas_call(
        _tca2_body,
        out_shape=(
            jax.ShapeDtypeStruct((N, H1), jnp.float32),
            jax.ShapeDtypeStruct((N, 1), jnp.float32),
        ),
    )


def _bn_relu(y, g, b):
    m = jnp.mean(y, axis=0, keepdims=True)
    v = jnp.mean((y - m) ** 2, axis=0, keepdims=True)
    return jnp.maximum((y - m) * lax.rsqrt(v + 1e-5) * g + b, 0.0)


def _tcb_body(s1_ref, dinv_ref, b1_ref, g1_ref, be1_ref, w2_ref,
              hp2_ref):
    dinv = dinv_ref[...]
    acc = s1_ref[0] + s1_ref[1]
    out1 = dinv * acc + b1_ref[...]
    h = _bn_relu(out1, g1_ref[...], be1_ref[...])
    xw2 = jnp.dot(h, w2_ref[...], preferred_element_type=jnp.float32)
    hp2_ref[...] = xw2 * dinv


@functools.lru_cache(maxsize=None)
def _make_tcb():
    return pl.pallas_call(
        _tcb_body,
        out_shape=jax.ShapeDtypeStruct((N, H2), jnp.float32),
    )


def _tcc_body(s2_ref, dinv_ref, b2_ref, g2_ref, be2_ref, batch_ref,
              eps_ref, wmu_ref, bmu_ref, wlv_ref, blv_ref, wd1_ref, bd1_ref,
              wd2_ref, bd2_ref, wd3_ref, bd3_ref, wc1_ref, bc1_ref, wc2_ref,
              bc2_ref, cls_ref, recon_ref, mu_ref, lv_ref, z_ref):
    dinv = dinv_ref[...]
    acc = s2_ref[0] + s2_ref[1]
    out2 = dinv * acc + b2_ref[...]
    h = _bn_relu(out2, g2_ref[...], be2_ref[...])

    gid = lax.broadcasted_iota(jnp.int32, (1, NG), 1)
    P = (batch_ref[...] == gid).astype(jnp.float32)
    dims = (((0,), (0,)), ((), ()))
    sums = lax.dot_general(P, h, dims, preferred_element_type=jnp.float32)
    cnt = lax.dot_general(P, jnp.ones((N, 1), jnp.float32), dims,
                          preferred_element_type=jnp.float32)
    hg = sums / jnp.maximum(cnt, 1.0)

    mu = jnp.dot(hg, wmu_ref[...], preferred_element_type=jnp.float32) + bmu_ref[...]
    lv = jnp.dot(hg, wlv_ref[...], preferred_element_type=jnp.float32) + blv_ref[...]
    std = jnp.exp(0.5 * lv)
    z = mu + eps_ref[...] * std
    d = jnp.maximum(jnp.dot(z, wd1_ref[...], preferred_element_type=jnp.float32) + bd1_ref[...], 0.0)
    d = jnp.maximum(jnp.dot(d, wd2_ref[...], preferred_element_type=jnp.float32) + bd2_ref[...], 0.0)
    recon = jnp.dot(d, wd3_ref[...], preferred_element_type=jnp.float32) + bd3_ref[...]
    c = jnp.maximum(jnp.dot(z, wc1_ref[...], preferred_element_type=jnp.float32) + bc1_ref[...], 0.0)
    cls = jnp.dot(c, wc2_ref[...], preferred_element_type=jnp.float32) + bc2_ref[...]

    cls_ref[...] = cls
    recon_ref[...] = recon
    mu_ref[...] = mu
    lv_ref[...] = lv
    z_ref[...] = z


@functools.lru_cache(maxsize=None)
def _make_tcc():
    return pl.pallas_call(
        _tcc_body,
        out_shape=(
            jax.ShapeDtypeStruct((NG, NCLS), jnp.float32),
            jax.ShapeDtypeStruct((NG, D_IN), jnp.float32),
            jax.ShapeDtypeStruct((NG, LAT), jnp.float32),
            jax.ShapeDtypeStruct((NG, LAT), jnp.float32),
            jax.ShapeDtypeStruct((NG, LAT), jnp.float32),
        ),
    )


def _shape_edges(row):
    return row.reshape(NW, NCH, CHUNK)


def kernel(x, edge_index, batch, W1, b1, g1, be1, W2, b2, g2, be2, Wmu, bmu,
           Wlv, blv, Wd1, bd1, Wd2, bd2, Wd3, bd3, Wc1, bc1, Wc2, bc2):
    f32 = jnp.float32
    src = _shape_edges(edge_index[0])
    dst = _shape_edges(edge_index[1])

    deg_parts = _make_deg()(dst, jnp.zeros((N,), f32))
    xw1 = _make_tca1()(x, W1)
    hp1, dinv = _make_tca2()(deg_parts, xw1)

    s1 = _make_prop(H1)(hp1, src, dst, jnp.zeros((RPT, H1), f32))
    hp2 = _make_tcb()(s1, dinv, b1.reshape(1, H1),
                      g1.reshape(1, H1), be1.reshape(1, H1), W2)

    s2 = _make_prop(H2)(hp2, src, dst, jnp.zeros((RPT, H2), f32))

    cls, recon, mu, lv, z = _make_tcc()(
        s2, dinv, b2.reshape(1, H2), g2.reshape(1, H2),
        be2.reshape(1, H2), batch.reshape(N, 1),
        jax.random.normal(jax.random.key(42), (NG, LAT), f32),
        Wmu, bmu.reshape(1, LAT), Wlv, blv.reshape(1, LAT),
        Wd1, bd1.reshape(1, H2), Wd2, bd2.reshape(1, H2),
        Wd3, bd3.reshape(1, D_IN), Wc1, bc1.reshape(1, LAT // 2),
        Wc2, bc2.reshape(1, NCLS))
    return (cls, recon, mu, lv, z)


# speedup vs baseline: 2.1347x; 1.0084x over previous
import functools

import jax
import jax.numpy as jnp
import numpy as np
from jax import lax
from jax.experimental import pallas as pl
from jax.experimental.pallas import tpu as pltpu
from jax.experimental.pallas import tpu_sc as plsc

N = 10000
E = 320000
D_IN = 128
H1 = 64
H2 = 32
LAT = 128
NCLS = 2
NG = 16

NCORE = 2
NSUB = 16
LANES = 16
NW = NCORE * NSUB
EW = E // NW
CHUNK = 80
NCH = EW // CHUNK
RPT = N // NSUB
NBUF = 5


def _sc_mesh():
    return plsc.VectorSubcoreMesh(core_axis_name="c", subcore_axis_name="s")


_SC_PARAMS = dict(compiler_params=pltpu.CompilerParams(
    needs_layout_passes=False, use_tc_tiling_on_sc=False))


def _deg_body(dst_hbm, zeros_hbm, out_hbm, idx_v, deg_v):
    cid = lax.axis_index("c")
    sid = lax.axis_index("s")
    wid = sid * NCORE + cid
    pltpu.sync_copy(zeros_hbm, deg_v)
    pltpu.sync_copy(dst_hbm.at[wid], idx_v)
    ones = jnp.ones((LANES,), jnp.float32)

    def body(k, carry):
        for j in range(CHUNK // LANES):
            idx = idx_v[k, pl.ds(j * LANES, LANES)]
            plsc.addupdate_scatter(deg_v, [idx], ones)
        return carry

    lax.fori_loop(0, NCH, body, 0)
    pltpu.sync_copy(deg_v, out_hbm.at[wid])


@functools.lru_cache(maxsize=None)
def _make_deg():
    return pl.kernel(
        _deg_body,
        out_type=jax.ShapeDtypeStruct((NW, N), jnp.float32),
        mesh=_sc_mesh(),
        scratch_types=[
            pltpu.VMEM((NCH, CHUNK), jnp.int32),
            pltpu.VMEM((N,), jnp.float32),
        ],
        **_SC_PARAMS,
    )


def _prop_body(feat, table_hbm, src_hbm, dst_hbm, zeros_hbm, out_hbm,
               src_v, dst_v, b0, b1, b2, b3, b4, acc,
               s0, s1, s2, s3, s4):
    del feat
    bufs = (b0, b1, b2, b3, b4)
    sems = (s0, s1, s2, s3, s4)
    cid = lax.axis_index("c")
    sid = lax.axis_index("s")
    wid = sid * NCORE + cid
    @pl.when(cid == 0)
    def _():
        pltpu.sync_copy(table_hbm.at[pl.ds(sid * RPT, RPT)],
                        acc.at[pl.ds(sid * RPT, RPT)])

    @pl.when(cid == 1)
    def _():
        pltpu.sync_copy(zeros_hbm, acc.at[pl.ds(sid * RPT, RPT)])

    pltpu.sync_copy(src_hbm.at[wid], src_v)
    pltpu.sync_copy(dst_hbm.at[wid], dst_v)
    plsc.subcore_barrier()

    for p in range(NBUF - 1):
        pltpu.async_copy(table_hbm.at[src_v.at[p]], bufs[p], sems[p])

    def outer(g, carry):
        for b in range(NBUF):
            k = g * NBUF + b
            pltpu.make_async_copy(table_hbm.at[src_v.at[k]],
                                  bufs[b], sems[b]).wait()
            nk = k + NBUF - 1
            slot = (b + NBUF - 1) % NBUF

            @pl.when(nk < NCH)
            def _():
                pltpu.async_copy(table_hbm.at[src_v.at[nk]],
                                 bufs[slot], sems[slot])

            pltpu.sync_copy(bufs[b], acc.at[dst_v.at[k]], add=True)
        return carry

    lax.fori_loop(0, NCH // NBUF, outer, 0)
    plsc.subcore_barrier()
    pltpu.sync_copy(acc.at[pl.ds(sid * RPT, RPT)],
                    out_hbm.at[cid, pl.ds(sid * RPT, RPT)])


@functools.lru_cache(maxsize=None)
def _make_prop(feat):
    return pl.kernel(
        functools.partial(_prop_body, feat),
        out_type=jax.ShapeDtypeStruct((NCORE, N, feat), jnp.float32),
        mesh=_sc_mesh(),
        scratch_types=[
            pltpu.VMEM((NCH, CHUNK), jnp.int32),
            pltpu.VMEM((NCH, CHUNK), jnp.int32),
        ] + [pltpu.VMEM((CHUNK, feat), jnp.float32) for _ in range(NBUF)]
          + [pltpu.VMEM_SHARED((N, feat), jnp.float32)]
          + [pltpu.SemaphoreType.DMA for _ in range(NBUF)],
        **_SC_PARAMS,
    )


def _tca_body(degp_ref, x_ref, w1_ref, hp_ref, dinv_ref):
    deg = lax.dot_general(degp_ref[...], jnp.ones((NW, 1), jnp.float32),
                          (((0,), (0,)), ((), ())),
                          preferred_element_type=jnp.float32) + 1.0
    dinv = lax.rsqrt(deg)
    xw = jnp.dot(x_ref[...], w1_ref[...], preferred_element_type=jnp.float32)
    hp_ref[...] = xw * dinv
    dinv_ref[...] = dinv


@functools.lru_cache(maxsize=None)
def _make_tca():
    return pl.pallas_call(
        _tca_body,
        out_shape=(
            jax.ShapeDtypeStruct((N, H1), jnp.float32),
            jax.ShapeDtypeStruct((N, 1), jnp.float32),
        ),
    )


def _bn_relu(y, g, b):
    m = jnp.mean(y, axis=0, keepdims=True)
    v = jnp.mean((y - m) ** 2, axis=0, keepdims=True)
    return jnp.maximum((y - m) * lax.rsqrt(v + 1e-5) * g + b, 0.0)


def _tcb_body(s1_ref, dinv_ref, b1_ref, g1_ref, be1_ref, w2_ref,
              hp2_ref):
    dinv = dinv_ref[...]
    acc = s1_ref[0] + s1_ref[1]
    out1 = dinv * acc + b1_ref[...]
    h = _bn_relu(out1, g1_ref[...], be1_ref[...])
    xw2 = jnp.dot(h, w2_ref[...], preferred_element_type=jnp.float32)
    hp2_ref[...] = xw2 * dinv


@functools.lru_cache(maxsize=None)
def _make_tcb():
    return pl.pallas_call(
        _tcb_body,
        out_shape=jax.ShapeDtypeStruct((N, H2), jnp.float32),
    )


def _tcc_body(s2_ref, dinv_ref, b2_ref, g2_ref, be2_ref, batch_ref,
              eps_ref, wmu_ref, bmu_ref, wlv_ref, blv_ref, wd1_ref, bd1_ref,
              wd2_ref, bd2_ref, wd3_ref, bd3_ref, wc1_ref, bc1_ref, wc2_ref,
              bc2_ref, cls_ref, recon_ref, mu_ref, lv_ref, z_ref):
    dinv = dinv_ref[...]
    acc = s2_ref[0] + s2_ref[1]
    out2 = dinv * acc + b2_ref[...]
    h = _bn_relu(out2, g2_ref[...], be2_ref[...])

    gid = lax.broadcasted_iota(jnp.int32, (1, NG), 1)
    P = (batch_ref[...] == gid).astype(jnp.float32)
    dims = (((0,), (0,)), ((), ()))
    sums = lax.dot_general(P, h, dims, preferred_element_type=jnp.float32)
    cnt = lax.dot_general(P, jnp.ones((N, 1), jnp.float32), dims,
                          preferred_element_type=jnp.float32)
    hg = sums / jnp.maximum(cnt, 1.0)

    mu = jnp.dot(hg, wmu_ref[...], preferred_element_type=jnp.float32) + bmu_ref[...]
    lv = jnp.dot(hg, wlv_ref[...], preferred_element_type=jnp.float32) + blv_ref[...]
    std = jnp.exp(0.5 * lv)
    z = mu + eps_ref[...] * std
    d = jnp.maximum(jnp.dot(z, wd1_ref[...], preferred_element_type=jnp.float32) + bd1_ref[...], 0.0)
    d = jnp.maximum(jnp.dot(d, wd2_ref[...], preferred_element_type=jnp.float32) + bd2_ref[...], 0.0)
    recon = jnp.dot(d, wd3_ref[...], preferred_element_type=jnp.float32) + bd3_ref[...]
    c = jnp.maximum(jnp.dot(z, wc1_ref[...], preferred_element_type=jnp.float32) + bc1_ref[...], 0.0)
    cls = jnp.dot(c, wc2_ref[...], preferred_element_type=jnp.float32) + bc2_ref[...]

    cls_ref[...] = cls
    recon_ref[...] = recon
    mu_ref[...] = mu
    lv_ref[...] = lv
    z_ref[...] = z


@functools.lru_cache(maxsize=None)
def _make_tcc():
    return pl.pallas_call(
        _tcc_body,
        out_shape=(
            jax.ShapeDtypeStruct((NG, NCLS), jnp.float32),
            jax.ShapeDtypeStruct((NG, D_IN), jnp.float32),
            jax.ShapeDtypeStruct((NG, LAT), jnp.float32),
            jax.ShapeDtypeStruct((NG, LAT), jnp.float32),
            jax.ShapeDtypeStruct((NG, LAT), jnp.float32),
        ),
    )


def _shape_edges(row):
    return row.reshape(NW, NCH, CHUNK)


def kernel(x, edge_index, batch, W1, b1, g1, be1, W2, b2, g2, be2, Wmu, bmu,
           Wlv, blv, Wd1, bd1, Wd2, bd2, Wd3, bd3, Wc1, bc1, Wc2, bc2):
    f32 = jnp.float32
    src = _shape_edges(edge_index[0])
    dst = _shape_edges(edge_index[1])

    deg_parts = _make_deg()(dst, jnp.zeros((N,), f32))
    hp1, dinv = _make_tca()(deg_parts, x, W1)

    s1 = _make_prop(H1)(hp1, src, dst, jnp.zeros((RPT, H1), f32))
    hp2 = _make_tcb()(s1, dinv, b1.reshape(1, H1),
                      g1.reshape(1, H1), be1.reshape(1, H1), W2)

    s2 = _make_prop(H2)(hp2, src, dst, jnp.zeros((RPT, H2), f32))

    cls, recon, mu, lv, z = _make_tcc()(
        s2, dinv, b2.reshape(1, H2), g2.reshape(1, H2),
        be2.reshape(1, H2), batch.reshape(N, 1),
        jax.random.normal(jax.random.key(42), (NG, LAT), f32),
        Wmu, bmu.reshape(1, LAT), Wlv, blv.reshape(1, LAT),
        Wd1, bd1.reshape(1, H2), Wd2, bd2.reshape(1, H2),
        Wd3, bd3.reshape(1, D_IN), Wc1, bc1.reshape(1, LAT // 2),
        Wc2, bc2.reshape(1, NCLS))
    return (cls, recon, mu, lv, z)


# speedup vs baseline: 2.1604x; 1.0120x over previous
import functools

import jax
import jax.numpy as jnp
import numpy as np
from jax import lax
from jax.experimental import pallas as pl
from jax.experimental.pallas import tpu as pltpu
from jax.experimental.pallas import tpu_sc as plsc

N = 10000
E = 320000
D_IN = 128
H1 = 64
H2 = 32
LAT = 128
NCLS = 2
NG = 16

NCORE = 2
NSUB = 16
LANES = 16
NW = NCORE * NSUB
EW = E // NW
CHUNK = 80
NCH = EW // CHUNK
RPT = N // NSUB
def _nbuf(feat):
    return 5 if feat >= 64 else 25


def _sc_mesh():
    return plsc.VectorSubcoreMesh(core_axis_name="c", subcore_axis_name="s")


_SC_PARAMS = dict(compiler_params=pltpu.CompilerParams(
    needs_layout_passes=False, use_tc_tiling_on_sc=False))


def _deg_body(dst_hbm, zeros_hbm, out_hbm, idx_v, deg_v):
    cid = lax.axis_index("c")
    sid = lax.axis_index("s")
    wid = sid * NCORE + cid
    pltpu.sync_copy(zeros_hbm, deg_v)
    pltpu.sync_copy(dst_hbm.at[wid], idx_v)
    ones = jnp.ones((LANES,), jnp.float32)

    def body(k, carry):
        for j in range(CHUNK // LANES):
            idx = idx_v[k, pl.ds(j * LANES, LANES)]
            plsc.addupdate_scatter(deg_v, [idx], ones)
        return carry

    lax.fori_loop(0, NCH, body, 0)
    pltpu.sync_copy(deg_v, out_hbm.at[wid])


@functools.lru_cache(maxsize=None)
def _make_deg():
    return pl.kernel(
        _deg_body,
        out_type=jax.ShapeDtypeStruct((NW, N), jnp.float32),
        mesh=_sc_mesh(),
        scratch_types=[
            pltpu.VMEM((NCH, CHUNK), jnp.int32),
            pltpu.VMEM((N,), jnp.float32),
        ],
        **_SC_PARAMS,
    )


def _prop_body(feat, table_hbm, src_hbm, dst_hbm, zeros_hbm, out_hbm,
               src_v, dst_v, *rest):
    nbuf = _nbuf(feat)
    bufs = rest[:nbuf]
    acc = rest[nbuf]
    sems = rest[nbuf + 1:]
    cid = lax.axis_index("c")
    sid = lax.axis_index("s")
    wid = sid * NCORE + cid
    @pl.when(cid == 0)
    def _():
        pltpu.sync_copy(table_hbm.at[pl.ds(sid * RPT, RPT)],
                        acc.at[pl.ds(sid * RPT, RPT)])

    @pl.when(cid == 1)
    def _():
        pltpu.sync_copy(zeros_hbm, acc.at[pl.ds(sid * RPT, RPT)])

    pltpu.sync_copy(src_hbm.at[wid], src_v)
    pltpu.sync_copy(dst_hbm.at[wid], dst_v)
    plsc.subcore_barrier()

    for p in range(nbuf - 1):
        pltpu.async_copy(table_hbm.at[src_v.at[p]], bufs[p], sems[p])

    def outer(g, carry):
        for b in range(nbuf):
            k = g * nbuf + b
            pltpu.make_async_copy(table_hbm.at[src_v.at[k]],
                                  bufs[b], sems[b]).wait()
            nk = k + nbuf - 1
            slot = (b + nbuf - 1) % nbuf

            @pl.when(nk < NCH)
            def _():
                pltpu.async_copy(table_hbm.at[src_v.at[nk]],
                                 bufs[slot], sems[slot])

            pltpu.sync_copy(bufs[b], acc.at[dst_v.at[k]], add=True)
        return carry

    lax.fori_loop(0, NCH // nbuf, outer, 0)
    plsc.subcore_barrier()
    pltpu.sync_copy(acc.at[pl.ds(sid * RPT, RPT)],
                    out_hbm.at[cid, pl.ds(sid * RPT, RPT)])


@functools.lru_cache(maxsize=None)
def _make_prop(feat):
    return pl.kernel(
        functools.partial(_prop_body, feat),
        out_type=jax.ShapeDtypeStruct((NCORE, N, feat), jnp.float32),
        mesh=_sc_mesh(),
        scratch_types=[
            pltpu.VMEM((NCH, CHUNK), jnp.int32),
            pltpu.VMEM((NCH, CHUNK), jnp.int32),
        ] + [pltpu.VMEM((CHUNK, feat), jnp.float32)
             for _ in range(_nbuf(feat))]
          + [pltpu.VMEM_SHARED((N, feat), jnp.float32)]
          + [pltpu.SemaphoreType.DMA for _ in range(_nbuf(feat))],
        **_SC_PARAMS,
    )


def _tca_body(degp_ref, x_ref, w1_ref, hp_ref, dinv_ref):
    deg = lax.dot_general(degp_ref[...], jnp.ones((NW, 1), jnp.float32),
                          (((0,), (0,)), ((), ())),
                          preferred_element_type=jnp.float32) + 1.0
    dinv = lax.rsqrt(deg)
    xw = jnp.dot(x_ref[...], w1_ref[...], preferred_element_type=jnp.float32)
    hp_ref[...] = xw * dinv
    dinv_ref[...] = dinv


@functools.lru_cache(maxsize=None)
def _make_tca():
    return pl.pallas_call(
        _tca_body,
        out_shape=(
            jax.ShapeDtypeStruct((N, H1), jnp.float32),
            jax.ShapeDtypeStruct((N, 1), jnp.float32),
        ),
    )


def _bn_relu(y, g, b):
    m = jnp.mean(y, axis=0, keepdims=True)
    v = jnp.mean((y - m) ** 2, axis=0, keepdims=True)
    return jnp.maximum((y - m) * lax.rsqrt(v + 1e-5) * g + b, 0.0)


def _tcb_body(s1_ref, dinv_ref, b1_ref, g1_ref, be1_ref, w2_ref,
              hp2_ref):
    dinv = dinv_ref[...]
    acc = s1_ref[0] + s1_ref[1]
    out1 = dinv * acc + b1_ref[...]
    h = _bn_relu(out1, g1_ref[...], be1_ref[...])
    xw2 = jnp.dot(h, w2_ref[...], preferred_element_type=jnp.float32)
    hp2_ref[...] = xw2 * dinv


@functools.lru_cache(maxsize=None)
def _make_tcb():
    return pl.pallas_call(
        _tcb_body,
        out_shape=jax.ShapeDtypeStruct((N, H2), jnp.float32),
    )


def _tcc_body(s2_ref, dinv_ref, b2_ref, g2_ref, be2_ref, batch_ref,
              eps_ref, wmu_ref, bmu_ref, wlv_ref, blv_ref, wd1_ref, bd1_ref,
              wd2_ref, bd2_ref, wd3_ref, bd3_ref, wc1_ref, bc1_ref, wc2_ref,
              bc2_ref, cls_ref, recon_ref, mu_ref, lv_ref, z_ref):
    dinv = dinv_ref[...]
    acc = s2_ref[0] + s2_ref[1]
    out2 = dinv * acc + b2_ref[...]
    h = _bn_relu(out2, g2_ref[...], be2_ref[...])

    gid = lax.broadcasted_iota(jnp.int32, (1, NG), 1)
    P = (batch_ref[...] == gid).astype(jnp.float32)
    dims = (((0,), (0,)), ((), ()))
    sums = lax.dot_general(P, h, dims, preferred_element_type=jnp.float32)
    cnt = lax.dot_general(P, jnp.ones((N, 1), jnp.float32), dims,
                          preferred_element_type=jnp.float32)
    hg = sums / jnp.maximum(cnt, 1.0)

    mu = jnp.dot(hg, wmu_ref[...], preferred_element_type=jnp.float32) + bmu_ref[...]
    lv = jnp.dot(hg, wlv_ref[...], preferred_element_type=jnp.float32) + blv_ref[...]
    std = jnp.exp(0.5 * lv)
    z = mu + eps_ref[...] * std
    d = jnp.maximum(jnp.dot(z, wd1_ref[...], preferred_element_type=jnp.float32) + bd1_ref[...], 0.0)
    d = jnp.maximum(jnp.dot(d, wd2_ref[...], preferred_element_type=jnp.float32) + bd2_ref[...], 0.0)
    recon = jnp.dot(d, wd3_ref[...], preferred_element_type=jnp.float32) + bd3_ref[...]
    c = jnp.maximum(jnp.dot(z, wc1_ref[...], preferred_element_type=jnp.float32) + bc1_ref[...], 0.0)
    cls = jnp.dot(c, wc2_ref[...], preferred_element_type=jnp.float32) + bc2_ref[...]

    cls_ref[...] = cls
    recon_ref[...] = recon
    mu_ref[...] = mu
    lv_ref[...] = lv
    z_ref[...] = z


@functools.lru_cache(maxsize=None)
def _make_tcc():
    return pl.pallas_call(
        _tcc_body,
        out_shape=(
            jax.ShapeDtypeStruct((NG, NCLS), jnp.float32),
            jax.ShapeDtypeStruct((NG, D_IN), jnp.float32),
            jax.ShapeDtypeStruct((NG, LAT), jnp.float32),
            jax.ShapeDtypeStruct((NG, LAT), jnp.float32),
            jax.ShapeDtypeStruct((NG, LAT), jnp.float32),
        ),
    )


def _shape_edges(row):
    return row.reshape(NW, NCH, CHUNK)


def kernel(x, edge_index, batch, W1, b1, g1, be1, W2, b2, g2, be2, Wmu, bmu,
           Wlv, blv, Wd1, bd1, Wd2, bd2, Wd3, bd3, Wc1, bc1, Wc2, bc2):
    f32 = jnp.float32
    src = _shape_edges(edge_index[0])
    dst = _shape_edges(edge_index[1])

    deg_parts = _make_deg()(dst, jnp.zeros((N,), f32))
    hp1, dinv = _make_tca()(deg_parts, x, W1)

    s1 = _make_prop(H1)(hp1, src, dst, jnp.zeros((RPT, H1), f32))
    hp2 = _make_tcb()(s1, dinv, b1.reshape(1, H1),
                      g1.reshape(1, H1), be1.reshape(1, H1), W2)

    s2 = _make_prop(H2)(hp2, src, dst, jnp.zeros((RPT, H2), f32))

    cls, recon, mu, lv, z = _make_tcc()(
        s2, dinv, b2.reshape(1, H2), g2.reshape(1, H2),
        be2.reshape(1, H2), batch.reshape(N, 1),
        jax.random.normal(jax.random.key(42), (NG, LAT), f32),
        Wmu, bmu.reshape(1, LAT), Wlv, blv.reshape(1, LAT),
        Wd1, bd1.reshape(1, H2), Wd2, bd2.reshape(1, H2),
        Wd3, bd3.reshape(1, D_IN), Wc1, bc1.reshape(1, LAT // 2),
        Wc2, bc2.reshape(1, NCLS))
    return (cls, recon, mu, lv, z)
